# Initial kernel scaffold; baseline (speedup 1.0000x reference)
import jax
import jax.numpy as jnp
from jax.experimental import pallas as pl


def kernel(x, relative_embedding):
    raise NotImplementedError("write your pallas kernel here")


# speedup vs baseline: 3.6773x; 3.6773x over previous
import jax
import jax.numpy as jnp
from jax.experimental import pallas as pl

_SEQ = 1024
_DIM = 64
_TBL = 2 * _SEQ - 1
_BI = 8


def _body(table_ref, x_ref, o_ref):
    i0 = pl.program_id(0) * _BI
    for r in range(_BI):
        win = table_ref[pl.ds(_SEQ - 1 - (i0 + r), _SEQ), :]
        o_ref[0, r] = x_ref[0, r] + win


def kernel(x, relative_embedding):
    rtable = relative_embedding[::-1]
    grid = (_SEQ // _BI,)
    return pl.pallas_call(
        _body,
        grid=grid,
        in_specs=[
            pl.BlockSpec((_TBL, _DIM), lambda i: (0, 0)),
            pl.BlockSpec((1, _BI, _SEQ, _DIM), lambda i: (0, i, 0, 0)),
        ],
        out_specs=pl.BlockSpec((1, _BI, _SEQ, _DIM), lambda i: (0, i, 0, 0)),
        out_shape=jax.ShapeDtypeStruct(x.shape, x.dtype),
    )(rtable, x)
